# baseline (speedup 1.0000x reference)
import jax
import jax.numpy as jnp
from jax import lax
from jax.experimental import pallas as pl
from jax.experimental.pallas import tpu as pltpu
from jax.experimental.pallas import tpu_sc as plsc

N = 10000
E = 320000
D = 128
H = 128
C = 2

NC = 2
NS = 16
DH = D // 2
EPT = E // NS
K = 400
CHUNKS = EPT // K
HALF = CHUNKS // 2
CW = 8
N_PAD = 10240
RPT = N_PAD // NS


def _seg_body(with_cnt, x_hbm, sa_hbm, ei_hbm, zrow_hbm, zcnt_hbm, ones_hbm,
              part_hbm, cntp_hbm, acc, cacc, sidx_a, sidx_b, didx,
              rows_a, rows_b, ones_v, gsem_a, gsem_b, isem_a, isem_b):
    c = lax.axis_index("c")
    s = lax.axis_index("s")

    pltpu.sync_copy(zrow_hbm, acc.at[pl.ds(s * RPT, RPT), :])
    if with_cnt:
        pltpu.sync_copy(zcnt_hbm, cacc.at[pl.ds(s * RPT, RPT), :])
        pltpu.sync_copy(ones_hbm, ones_v)

    base_w = s * EPT
    pltpu.sync_copy(ei_hbm.at[1].at[pl.ds(base_w, EPT)], didx)
    sa = sa_hbm.at[c]
    plsc.subcore_barrier()

    xs = x_hbm

    def start_idx(ci, ib, sem):
        pltpu.async_copy(sa.at[pl.ds(base_w + ci * K, K)], ib, sem)

    def wait_idx(ci, ib, sem):
        pltpu.make_async_copy(sa.at[pl.ds(base_w + ci * K, K)], ib,
                              sem).wait()

    def start_gather(ib, rb, sem):
        pltpu.async_copy(xs.at[ib], rb, sem)

    def wait_gather(ib, rb, sem):
        pltpu.make_async_copy(xs.at[ib], rb, sem).wait()

    def scatter(rb, ci):
        db = didx.at[pl.ds(ci * K, K)]
        pltpu.sync_copy(rb, acc.at[db], add=True)
        if with_cnt:
            pltpu.sync_copy(ones_v, cacc.at[db], add=True)

    start_idx(0, sidx_a, isem_a)
    start_idx(1, sidx_b, isem_b)
    wait_idx(0, sidx_a, isem_a)
    start_gather(sidx_a, rows_a, gsem_a)
    wait_idx(1, sidx_b, isem_b)
    start_gather(sidx_b, rows_b, gsem_b)

    def step(j, carry):
        c0 = 2 * j
        wait_gather(sidx_a, rows_a, gsem_a)
        scatter(rows_a, c0)

        @pl.when(j + 1 < HALF)
        def _():
            start_idx(c0 + 2, sidx_a, isem_a)

        wait_gather(sidx_b, rows_b, gsem_b)
        scatter(rows_b, c0 + 1)

        @pl.when(j + 1 < HALF)
        def _():
            start_idx(c0 + 3, sidx_b, isem_b)
            wait_idx(c0 + 2, sidx_a, isem_a)
            start_gather(sidx_a, rows_a, gsem_a)
            wait_idx(c0 + 3, sidx_b, isem_b)
            start_gather(sidx_b, rows_b, gsem_b)

        return carry

    lax.fori_loop(0, HALF, step, 0)

    plsc.subcore_barrier()
    pltpu.sync_copy(acc.at[pl.ds(s * RPT, RPT), :],
                    part_hbm.at[c, pl.ds(s * RPT, RPT), :])
    if with_cnt:
        pltpu.sync_copy(cacc.at[pl.ds(s * RPT, RPT), :],
                        cntp_hbm.at[c, pl.ds(s * RPT, RPT), :])


def _make_seg_sum(with_cnt):
    mesh = plsc.VectorSubcoreMesh(core_axis_name="c", subcore_axis_name="s")
    out_type = [jax.ShapeDtypeStruct((NC, N_PAD, DH), jnp.float32)]
    scratch = [
        pltpu.VMEM_SHARED((N_PAD, DH), jnp.float32),
        pltpu.VMEM((K,), jnp.int32),
        pltpu.VMEM((K,), jnp.int32),
        pltpu.VMEM((EPT,), jnp.int32),
        pltpu.VMEM((K, DH), jnp.float32),
        pltpu.VMEM((K, DH), jnp.float32),
        pltpu.SemaphoreType.DMA,
        pltpu.SemaphoreType.DMA,
        pltpu.SemaphoreType.DMA,
        pltpu.SemaphoreType.DMA,
    ]
    if with_cnt:
        out_type.append(jax.ShapeDtypeStruct((NC, N_PAD, CW), jnp.float32))
        scratch.insert(1, pltpu.VMEM_SHARED((N_PAD, CW), jnp.float32))
        scratch.insert(-4, pltpu.VMEM((K, CW), jnp.float32))

    if with_cnt:
        def body(x_hbm, sa_hbm, ei_hbm, zrow_hbm, zcnt_hbm, ones_hbm,
                 part_hbm, cntp_hbm, acc, cacc, sidx_a, sidx_b, didx,
                 rows_a, rows_b, ones_v, gsem_a, gsem_b, isem_a, isem_b):
            _seg_body(True, x_hbm, sa_hbm, ei_hbm, zrow_hbm, zcnt_hbm,
                      ones_hbm, part_hbm, cntp_hbm, acc, cacc, sidx_a,
                      sidx_b, didx, rows_a, rows_b, ones_v, gsem_a, gsem_b,
                      isem_a, isem_b)
    else:
        def body(x_hbm, sa_hbm, ei_hbm, zrow_hbm, part_hbm, acc,
                 sidx_a, sidx_b, didx, rows_a, rows_b, gsem_a, gsem_b,
                 isem_a, isem_b):
            _seg_body(False, x_hbm, sa_hbm, ei_hbm, zrow_hbm, None,
                      None, part_hbm, None, acc, None, sidx_a, sidx_b,
                      didx, rows_a, rows_b, None, gsem_a, gsem_b,
                      isem_a, isem_b)

    return pl.kernel(body, out_type=tuple(out_type), mesh=mesh,
                     scratch_types=scratch,
                     compiler_params=pltpu.CompilerParams(
                         use_tc_tiling_on_sc=False))


_seg_sum_cnt = _make_seg_sum(True)
_seg_sum = _make_seg_sum(False)

BR = 2000


def _tc1_body(x_ref, p_ref, c_ref, w_ref, b_ref, h_ref):
    cnt = c_ref[0, :, 0:1]
    recip = 1.0 / jnp.maximum(cnt, 1.0)
    h = jnp.dot(x_ref[...], w_ref[:D, :], preferred_element_type=jnp.float32)
    h += jnp.dot(p_ref[0] * recip, w_ref[D:D + DH, :],
                 preferred_element_type=jnp.float32)
    h += jnp.dot(p_ref[1] * recip, w_ref[D + DH:, :],
                 preferred_element_type=jnp.float32)
    h += b_ref[...]
    h_ref[...] = jnp.maximum(h, 0.0)


def _tc2_body(h_ref, p_ref, c_ref, w_ref, b_ref, wl1_ref, bl1_ref,
              wl2_ref, bl2_ref, out_ref):
    cnt = c_ref[0, :, 0:1]
    recip = 1.0 / jnp.maximum(cnt, 1.0)
    h2 = jnp.dot(h_ref[...], w_ref[:H, :], preferred_element_type=jnp.float32)
    h2 += jnp.dot(p_ref[0] * recip, w_ref[H:H + DH, :],
                  preferred_element_type=jnp.float32)
    h2 += jnp.dot(p_ref[1] * recip, w_ref[H + DH:, :],
                  preferred_element_type=jnp.float32)
    h2 += b_ref[...]
    h2 = jnp.maximum(h2, 0.0)
    s = jnp.dot(h2, wl1_ref[...], preferred_element_type=jnp.float32)
    s = jnp.maximum(s + bl1_ref[...], 0.0)
    out = jnp.dot(s, wl2_ref[...], preferred_element_type=jnp.float32)
    out_ref[...] = out + bl2_ref[...]


def _tc1(x, P, Cn, W1, b1):
    return pl.pallas_call(
        _tc1_body,
        grid=(N // BR,),
        in_specs=[
            pl.BlockSpec((BR, D), lambda i: (i, 0)),
            pl.BlockSpec((NC, BR, DH), lambda i: (0, i, 0)),
            pl.BlockSpec((1, BR, CW), lambda i: (0, i, 0)),
            pl.BlockSpec((2 * D, H), lambda i: (0, 0)),
            pl.BlockSpec((1, H), lambda i: (0, 0)),
        ],
        out_specs=pl.BlockSpec((BR, H), lambda i: (i, 0)),
        out_shape=jax.ShapeDtypeStruct((N, H), jnp.float32),
    )(x, P, Cn, W1, b1)


def _tc2(hs, P, Cn, W2, b2, Wl1, bl1, Wl2, bl2):
    return pl.pallas_call(
        _tc2_body,
        grid=(N // BR,),
        in_specs=[
            pl.BlockSpec((BR, H), lambda i: (i, 0)),
            pl.BlockSpec((NC, BR, DH), lambda i: (0, i, 0)),
            pl.BlockSpec((1, BR, CW), lambda i: (0, i, 0)),
            pl.BlockSpec((2 * H, H), lambda i: (0, 0)),
            pl.BlockSpec((1, H), lambda i: (0, 0)),
            pl.BlockSpec((H, H), lambda i: (0, 0)),
            pl.BlockSpec((1, H), lambda i: (0, 0)),
            pl.BlockSpec((H, C), lambda i: (0, 0)),
            pl.BlockSpec((1, C), lambda i: (0, 0)),
        ],
        out_specs=pl.BlockSpec((BR, C), lambda i: (i, 0)),
        out_shape=jax.ShapeDtypeStruct((N, C), jnp.float32),
    )(hs, P, Cn, W2, b2, Wl1, bl1, Wl2, bl2)


def kernel(x, edge_index, W1, b1, W2, b2, Wl1, bl1, Wl2, bl2):
    xv = x.reshape(2 * N, DH)

    zrow = jnp.zeros((RPT, DH), jnp.float32)
    zcnt = jnp.zeros((RPT, CW), jnp.float32)
    ones = jnp.ones((K, CW), jnp.float32)

    src2 = edge_index[0] * 2
    srcadj = jnp.stack([src2, src2 + 1])

    P1, Cn = _seg_sum_cnt(xv, srcadj, edge_index, zrow, zcnt, ones)
    h = _tc1(x, P1, Cn, W1, b1.reshape(1, H))

    P2 = _seg_sum(h.reshape(2 * N, DH), srcadj, edge_index, zrow)
    P2 = P2[0] if isinstance(P2, (list, tuple)) else P2

    return _tc2(h, P2, Cn, W2, b2.reshape(1, H), Wl1, bl1.reshape(1, H),
                Wl2, bl2.reshape(1, C))


# speedup vs baseline: 1.0632x; 1.0632x over previous
import jax
import jax.numpy as jnp
from jax import lax
from jax.experimental import pallas as pl
from jax.experimental.pallas import tpu as pltpu
from jax.experimental.pallas import tpu_sc as plsc

N = 10000
E = 320000
D = 128
H = 128
C = 2

NC = 2
NS = 16
DH = D // 2
EPT = E // NS
K = 200
CHUNKS = EPT // K
HALF = CHUNKS // 2
CW = 8
N_PAD = 10240
RPT = N_PAD // NS


def _seg_body(with_cnt, x_hbm, sa_hbm, ei_hbm, zrow_hbm, zcnt_hbm, ones_hbm,
              part_hbm, cntp_hbm, acc, cacc, sidx_a, sidx_b, didx,
              rows_a, rows_b, ones_v, gsem_a, gsem_b, isem_a, isem_b):
    c = lax.axis_index("c")
    s = lax.axis_index("s")

    pltpu.sync_copy(zrow_hbm, acc.at[pl.ds(s * RPT, RPT), :])
    if with_cnt:
        pltpu.sync_copy(zcnt_hbm, cacc.at[pl.ds(s * RPT, RPT), :])
        pltpu.sync_copy(ones_hbm, ones_v)

    base_w = s * EPT
    pltpu.sync_copy(ei_hbm.at[1].at[pl.ds(base_w, EPT)], didx)
    sa = sa_hbm.at[c]
    plsc.subcore_barrier()

    xs = x_hbm

    def start_idx(ci, ib, sem):
        pltpu.async_copy(sa.at[pl.ds(base_w + ci * K, K)], ib, sem)

    def wait_idx(ci, ib, sem):
        pltpu.make_async_copy(sa.at[pl.ds(base_w + ci * K, K)], ib,
                              sem).wait()

    def start_gather(ib, rb, sem):
        pltpu.async_copy(xs.at[ib], rb, sem)

    def wait_gather(ib, rb, sem):
        pltpu.make_async_copy(xs.at[ib], rb, sem).wait()

    def scatter(rb, ci):
        db = didx.at[pl.ds(ci * K, K)]
        pltpu.sync_copy(rb, acc.at[db], add=True)
        if with_cnt:
            pltpu.sync_copy(ones_v, cacc.at[db], add=True)

    start_idx(0, sidx_a, isem_a)
    start_idx(1, sidx_b, isem_b)
    wait_idx(0, sidx_a, isem_a)
    start_gather(sidx_a, rows_a, gsem_a)
    wait_idx(1, sidx_b, isem_b)
    start_gather(sidx_b, rows_b, gsem_b)

    def step(j, carry):
        c0 = 2 * j
        wait_gather(sidx_a, rows_a, gsem_a)
        scatter(rows_a, c0)

        @pl.when(j + 1 < HALF)
        def _():
            start_idx(c0 + 2, sidx_a, isem_a)
            wait_idx(c0 + 2, sidx_a, isem_a)
            start_gather(sidx_a, rows_a, gsem_a)

        wait_gather(sidx_b, rows_b, gsem_b)
        scatter(rows_b, c0 + 1)

        @pl.when(j + 1 < HALF)
        def _():
            start_idx(c0 + 3, sidx_b, isem_b)
            wait_idx(c0 + 3, sidx_b, isem_b)
            start_gather(sidx_b, rows_b, gsem_b)

        return carry

    lax.fori_loop(0, HALF, step, 0)

    plsc.subcore_barrier()
    pltpu.sync_copy(acc.at[pl.ds(s * RPT, RPT), :],
                    part_hbm.at[c, pl.ds(s * RPT, RPT), :])
    if with_cnt:
        pltpu.sync_copy(cacc.at[pl.ds(s * RPT, RPT), :],
                        cntp_hbm.at[c, pl.ds(s * RPT, RPT), :])


def _make_seg_sum(with_cnt):
    mesh = plsc.VectorSubcoreMesh(core_axis_name="c", subcore_axis_name="s")
    out_type = [jax.ShapeDtypeStruct((NC, N_PAD, DH), jnp.float32)]
    scratch = [
        pltpu.VMEM_SHARED((N_PAD, DH), jnp.float32),
        pltpu.VMEM((K,), jnp.int32),
        pltpu.VMEM((K,), jnp.int32),
        pltpu.VMEM((EPT,), jnp.int32),
        pltpu.VMEM((K, DH), jnp.float32),
        pltpu.VMEM((K, DH), jnp.float32),
        pltpu.SemaphoreType.DMA,
        pltpu.SemaphoreType.DMA,
        pltpu.SemaphoreType.DMA,
        pltpu.SemaphoreType.DMA,
    ]
    if with_cnt:
        out_type.append(jax.ShapeDtypeStruct((NC, N_PAD, CW), jnp.float32))
        scratch.insert(1, pltpu.VMEM_SHARED((N_PAD, CW), jnp.float32))
        scratch.insert(-4, pltpu.VMEM((K, CW), jnp.float32))

    if with_cnt:
        def body(x_hbm, sa_hbm, ei_hbm, zrow_hbm, zcnt_hbm, ones_hbm,
                 part_hbm, cntp_hbm, acc, cacc, sidx_a, sidx_b, didx,
                 rows_a, rows_b, ones_v, gsem_a, gsem_b, isem_a, isem_b):
            _seg_body(True, x_hbm, sa_hbm, ei_hbm, zrow_hbm, zcnt_hbm,
                      ones_hbm, part_hbm, cntp_hbm, acc, cacc, sidx_a,
                      sidx_b, didx, rows_a, rows_b, ones_v, gsem_a, gsem_b,
                      isem_a, isem_b)
    else:
        def body(x_hbm, sa_hbm, ei_hbm, zrow_hbm, part_hbm, acc,
                 sidx_a, sidx_b, didx, rows_a, rows_b, gsem_a, gsem_b,
                 isem_a, isem_b):
            _seg_body(False, x_hbm, sa_hbm, ei_hbm, zrow_hbm, None,
                      None, part_hbm, None, acc, None, sidx_a, sidx_b,
                      didx, rows_a, rows_b, None, gsem_a, gsem_b,
                      isem_a, isem_b)

    return pl.kernel(body, out_type=tuple(out_type), mesh=mesh,
                     scratch_types=scratch,
                     compiler_params=pltpu.CompilerParams(
                         use_tc_tiling_on_sc=False))


_seg_sum_cnt = _make_seg_sum(True)
_seg_sum = _make_seg_sum(False)

BR = 2000


def _tc1_body(x_ref, p_ref, c_ref, w_ref, b_ref, h_ref):
    cnt = c_ref[0, :, 0:1]
    recip = 1.0 / jnp.maximum(cnt, 1.0)
    h = jnp.dot(x_ref[...], w_ref[:D, :], preferred_element_type=jnp.float32)
    h += jnp.dot(p_ref[0] * recip, w_ref[D:D + DH, :],
                 preferred_element_type=jnp.float32)
    h += jnp.dot(p_ref[1] * recip, w_ref[D + DH:, :],
                 preferred_element_type=jnp.float32)
    h += b_ref[...]
    h_ref[...] = jnp.maximum(h, 0.0)


def _tc2_body(h_ref, p_ref, c_ref, w_ref, b_ref, wl1_ref, bl1_ref,
              wl2_ref, bl2_ref, out_ref):
    cnt = c_ref[0, :, 0:1]
    recip = 1.0 / jnp.maximum(cnt, 1.0)
    h2 = jnp.dot(h_ref[...], w_ref[:H, :], preferred_element_type=jnp.float32)
    h2 += jnp.dot(p_ref[0] * recip, w_ref[H:H + DH, :],
                  preferred_element_type=jnp.float32)
    h2 += jnp.dot(p_ref[1] * recip, w_ref[H + DH:, :],
                  preferred_element_type=jnp.float32)
    h2 += b_ref[...]
    h2 = jnp.maximum(h2, 0.0)
    s = jnp.dot(h2, wl1_ref[...], preferred_element_type=jnp.float32)
    s = jnp.maximum(s + bl1_ref[...], 0.0)
    out = jnp.dot(s, wl2_ref[...], preferred_element_type=jnp.float32)
    out_ref[...] = out + bl2_ref[...]


def _tc1(x, P, Cn, W1, b1):
    return pl.pallas_call(
        _tc1_body,
        grid=(N // BR,),
        in_specs=[
            pl.BlockSpec((BR, D), lambda i: (i, 0)),
            pl.BlockSpec((NC, BR, DH), lambda i: (0, i, 0)),
            pl.BlockSpec((1, BR, CW), lambda i: (0, i, 0)),
            pl.BlockSpec((2 * D, H), lambda i: (0, 0)),
            pl.BlockSpec((1, H), lambda i: (0, 0)),
        ],
        out_specs=pl.BlockSpec((BR, H), lambda i: (i, 0)),
        out_shape=jax.ShapeDtypeStruct((N, H), jnp.float32),
    )(x, P, Cn, W1, b1)


def _tc2(hs, P, Cn, W2, b2, Wl1, bl1, Wl2, bl2):
    return pl.pallas_call(
        _tc2_body,
        grid=(N // BR,),
        in_specs=[
            pl.BlockSpec((BR, H), lambda i: (i, 0)),
            pl.BlockSpec((NC, BR, DH), lambda i: (0, i, 0)),
            pl.BlockSpec((1, BR, CW), lambda i: (0, i, 0)),
            pl.BlockSpec((2 * H, H), lambda i: (0, 0)),
            pl.BlockSpec((1, H), lambda i: (0, 0)),
            pl.BlockSpec((H, H), lambda i: (0, 0)),
            pl.BlockSpec((1, H), lambda i: (0, 0)),
            pl.BlockSpec((H, C), lambda i: (0, 0)),
            pl.BlockSpec((1, C), lambda i: (0, 0)),
        ],
        out_specs=pl.BlockSpec((BR, C), lambda i: (i, 0)),
        out_shape=jax.ShapeDtypeStruct((N, C), jnp.float32),
    )(hs, P, Cn, W2, b2, Wl1, bl1, Wl2, bl2)


def kernel(x, edge_index, W1, b1, W2, b2, Wl1, bl1, Wl2, bl2):
    xv = x.reshape(2 * N, DH)

    zrow = jnp.zeros((RPT, DH), jnp.float32)
    zcnt = jnp.zeros((RPT, CW), jnp.float32)
    ones = jnp.ones((K, CW), jnp.float32)

    src2 = edge_index[0] * 2
    srcadj = jnp.stack([src2, src2 + 1])

    P1, Cn = _seg_sum_cnt(xv, srcadj, edge_index, zrow, zcnt, ones)
    h = _tc1(x, P1, Cn, W1, b1.reshape(1, H))

    P2 = _seg_sum(h.reshape(2 * N, DH), srcadj, edge_index, zrow)
    P2 = P2[0] if isinstance(P2, (list, tuple)) else P2

    return _tc2(h, P2, Cn, W2, b2.reshape(1, H), Wl1, bl1.reshape(1, H),
                Wl2, bl2.reshape(1, C))


# speedup vs baseline: 1.2219x; 1.1493x over previous
import jax
import jax.numpy as jnp
from jax import lax
from jax.experimental import pallas as pl
from jax.experimental.pallas import tpu as pltpu
from jax.experimental.pallas import tpu_sc as plsc

N = 10000
E = 320000
D = 128
H = 128
C = 2

NC = 2
NS = 16
DH = D // 2
EPT = E // NS
K = 200
CHUNKS = EPT // K
HALF = CHUNKS // 2
CW = 8
N_PAD = 10240
RPT = N_PAD // NS


def _seg_body(with_cnt, x_hbm, sa_hbm, ei_hbm, zrow_hbm, zcnt_hbm, ones_hbm,
              part_hbm, cntp_hbm, acc, cacc, sidx, didx, rows_a, rows_b,
              ones_v, gsem_a, gsem_b, isem_a, isem_b):
    c = lax.axis_index("c")
    s = lax.axis_index("s")

    pltpu.sync_copy(zrow_hbm, acc.at[pl.ds(s * RPT, RPT), :])
    if with_cnt:
        pltpu.sync_copy(zcnt_hbm, cacc.at[pl.ds(s * RPT, RPT), :])
        pltpu.sync_copy(ones_hbm, ones_v)

    base_w = s * EPT
    pltpu.sync_copy(sa_hbm.at[c].at[pl.ds(base_w, EPT)], sidx)
    pltpu.sync_copy(ei_hbm.at[1].at[pl.ds(base_w, EPT)], didx)
    plsc.subcore_barrier()

    xs = x_hbm

    def start_gather(ci, rb, sem):
        pltpu.async_copy(xs.at[sidx.at[pl.ds(ci * K, K)]], rb, sem)

    def wait_gather(ci, rb, sem):
        pltpu.make_async_copy(xs.at[sidx.at[pl.ds(ci * K, K)]], rb,
                              sem).wait()

    def scatter(rb, ci):
        db = didx.at[pl.ds(ci * K, K)]
        pltpu.sync_copy(rb, acc.at[db], add=True)
        if with_cnt:
            pltpu.sync_copy(ones_v, cacc.at[db], add=True)

    start_gather(0, rows_a, gsem_a)
    start_gather(1, rows_b, gsem_b)

    def step(j, carry):
        c0 = 2 * j
        wait_gather(c0, rows_a, gsem_a)
        scatter(rows_a, c0)

        @pl.when(j + 1 < HALF)
        def _():
            start_gather(c0 + 2, rows_a, gsem_a)

        wait_gather(c0 + 1, rows_b, gsem_b)
        scatter(rows_b, c0 + 1)

        @pl.when(j + 1 < HALF)
        def _():
            start_gather(c0 + 3, rows_b, gsem_b)

        return carry

    lax.fori_loop(0, HALF, step, 0)

    plsc.subcore_barrier()
    pltpu.sync_copy(acc.at[pl.ds(s * RPT, RPT), :],
                    part_hbm.at[c, pl.ds(s * RPT, RPT), :])
    if with_cnt:
        pltpu.sync_copy(cacc.at[pl.ds(s * RPT, RPT), :],
                        cntp_hbm.at[c, pl.ds(s * RPT, RPT), :])


def _make_seg_sum(with_cnt):
    mesh = plsc.VectorSubcoreMesh(core_axis_name="c", subcore_axis_name="s")
    out_type = [jax.ShapeDtypeStruct((NC, N_PAD, DH), jnp.float32)]
    scratch = [
        pltpu.VMEM_SHARED((N_PAD, DH), jnp.float32),
        pltpu.VMEM((EPT,), jnp.int32),
        pltpu.VMEM((EPT,), jnp.int32),
        pltpu.VMEM((K, DH), jnp.float32),
        pltpu.VMEM((K, DH), jnp.float32),
        pltpu.SemaphoreType.DMA,
        pltpu.SemaphoreType.DMA,
        pltpu.SemaphoreType.DMA,
        pltpu.SemaphoreType.DMA,
    ]
    if with_cnt:
        out_type.append(jax.ShapeDtypeStruct((NC, N_PAD, CW), jnp.float32))
        scratch.insert(1, pltpu.VMEM_SHARED((N_PAD, CW), jnp.float32))
        scratch.insert(-4, pltpu.VMEM((K, CW), jnp.float32))

    if with_cnt:
        def body(x_hbm, sa_hbm, ei_hbm, zrow_hbm, zcnt_hbm, ones_hbm,
                 part_hbm, cntp_hbm, acc, cacc, sidx, didx,
                 rows_a, rows_b, ones_v, gsem_a, gsem_b, isem_a, isem_b):
            _seg_body(True, x_hbm, sa_hbm, ei_hbm, zrow_hbm, zcnt_hbm,
                      ones_hbm, part_hbm, cntp_hbm, acc, cacc, sidx,
                      didx, rows_a, rows_b, ones_v, gsem_a, gsem_b,
                      isem_a, isem_b)
    else:
        def body(x_hbm, sa_hbm, ei_hbm, zrow_hbm, part_hbm, acc,
                 sidx, didx, rows_a, rows_b, gsem_a, gsem_b,
                 isem_a, isem_b):
            _seg_body(False, x_hbm, sa_hbm, ei_hbm, zrow_hbm, None,
                      None, part_hbm, None, acc, None, sidx, didx,
                      rows_a, rows_b, None, gsem_a, gsem_b,
                      isem_a, isem_b)

    return pl.kernel(body, out_type=tuple(out_type), mesh=mesh,
                     scratch_types=scratch,
                     compiler_params=pltpu.CompilerParams(
                         use_tc_tiling_on_sc=False))


_seg_sum_cnt = _make_seg_sum(True)
_seg_sum = _make_seg_sum(False)

BR = 2000


def _tc1_body(x_ref, p_ref, c_ref, w_ref, b_ref, h_ref):
    cnt = c_ref[0, :, 0:1]
    recip = 1.0 / jnp.maximum(cnt, 1.0)
    h = jnp.dot(x_ref[...], w_ref[:D, :], preferred_element_type=jnp.float32)
    h += jnp.dot(p_ref[0] * recip, w_ref[D:D + DH, :],
                 preferred_element_type=jnp.float32)
    h += jnp.dot(p_ref[1] * recip, w_ref[D + DH:, :],
                 preferred_element_type=jnp.float32)
    h += b_ref[...]
    h_ref[...] = jnp.maximum(h, 0.0)


def _tc2_body(h_ref, p_ref, c_ref, w_ref, b_ref, wl1_ref, bl1_ref,
              wl2_ref, bl2_ref, out_ref):
    cnt = c_ref[0, :, 0:1]
    recip = 1.0 / jnp.maximum(cnt, 1.0)
    h2 = jnp.dot(h_ref[...], w_ref[:H, :], preferred_element_type=jnp.float32)
    h2 += jnp.dot(p_ref[0] * recip, w_ref[H:H + DH, :],
                  preferred_element_type=jnp.float32)
    h2 += jnp.dot(p_ref[1] * recip, w_ref[H + DH:, :],
                  preferred_element_type=jnp.float32)
    h2 += b_ref[...]
    h2 = jnp.maximum(h2, 0.0)
    s = jnp.dot(h2, wl1_ref[...], preferred_element_type=jnp.float32)
    s = jnp.maximum(s + bl1_ref[...], 0.0)
    out = jnp.dot(s, wl2_ref[...], preferred_element_type=jnp.float32)
    out_ref[...] = out + bl2_ref[...]


def _tc1(x, P, Cn, W1, b1):
    return pl.pallas_call(
        _tc1_body,
        grid=(N // BR,),
        in_specs=[
            pl.BlockSpec((BR, D), lambda i: (i, 0)),
            pl.BlockSpec((NC, BR, DH), lambda i: (0, i, 0)),
            pl.BlockSpec((1, BR, CW), lambda i: (0, i, 0)),
            pl.BlockSpec((2 * D, H), lambda i: (0, 0)),
            pl.BlockSpec((1, H), lambda i: (0, 0)),
        ],
        out_specs=pl.BlockSpec((BR, H), lambda i: (i, 0)),
        out_shape=jax.ShapeDtypeStruct((N, H), jnp.float32),
    )(x, P, Cn, W1, b1)


def _tc2(hs, P, Cn, W2, b2, Wl1, bl1, Wl2, bl2):
    return pl.pallas_call(
        _tc2_body,
        grid=(N // BR,),
        in_specs=[
            pl.BlockSpec((BR, H), lambda i: (i, 0)),
            pl.BlockSpec((NC, BR, DH), lambda i: (0, i, 0)),
            pl.BlockSpec((1, BR, CW), lambda i: (0, i, 0)),
            pl.BlockSpec((2 * H, H), lambda i: (0, 0)),
            pl.BlockSpec((1, H), lambda i: (0, 0)),
            pl.BlockSpec((H, H), lambda i: (0, 0)),
            pl.BlockSpec((1, H), lambda i: (0, 0)),
            pl.BlockSpec((H, C), lambda i: (0, 0)),
            pl.BlockSpec((1, C), lambda i: (0, 0)),
        ],
        out_specs=pl.BlockSpec((BR, C), lambda i: (i, 0)),
        out_shape=jax.ShapeDtypeStruct((N, C), jnp.float32),
    )(hs, P, Cn, W2, b2, Wl1, bl1, Wl2, bl2)


def kernel(x, edge_index, W1, b1, W2, b2, Wl1, bl1, Wl2, bl2):
    xv = x.reshape(2 * N, DH)

    zrow = jnp.zeros((RPT, DH), jnp.float32)
    zcnt = jnp.zeros((RPT, CW), jnp.float32)
    ones = jnp.ones((K, CW), jnp.float32)

    src2 = edge_index[0] * 2
    srcadj = jnp.stack([src2, src2 + 1])

    P1, Cn = _seg_sum_cnt(xv, srcadj, edge_index, zrow, zcnt, ones)
    h = _tc1(x, P1, Cn, W1, b1.reshape(1, H))

    P2 = _seg_sum(h.reshape(2 * N, DH), srcadj, edge_index, zrow)
    P2 = P2[0] if isinstance(P2, (list, tuple)) else P2

    return _tc2(h, P2, Cn, W2, b2.reshape(1, H), Wl1, bl1.reshape(1, H),
                Wl2, bl2.reshape(1, C))
